# baseline (speedup 1.0000x reference)
import functools

import jax
import jax.numpy as jnp
from jax import lax
from jax.experimental import pallas as pl
from jax.experimental.pallas import tpu as pltpu
from jax.experimental.pallas import tpu_sc as plsc

N = 10000
E = 320000
D = 128
DE = 16
H = D // 2
NSUB = 16
K = 64
NCHT = E // K
BASECH = NCHT // NSUB
EXTRA = NCHT % NSUB
GROUPS = BASECH // 4
STRIPE = 624

_f32 = jnp.float32


def _prep_nodes_body(x_ref, wa_ref, ba_ref, wb_ref, bb_ref, wv_ref, bv_ref,
                     wu_ref, bu_ref,
                     sv0_ref, sv1_ref, b0_ref, b1_ref, u_ref):
    xv = x_ref[...]
    a = jnp.dot(xv, wa_ref[...], preferred_element_type=_f32) + ba_ref[...]
    b = jnp.dot(xv, wb_ref[...], preferred_element_type=_f32) + bb_ref[...]
    v = jnp.dot(xv, wv_ref[...], preferred_element_type=_f32) + bv_ref[...]
    u_ref[...] = jnp.dot(xv, wu_ref[...], preferred_element_type=_f32) + bu_ref[...]
    sv0_ref[...] = jnp.concatenate([a[:, :H], v[:, :H]], axis=1)
    sv1_ref[...] = jnp.concatenate([a[:, H:], v[:, H:]], axis=1)
    b0_ref[...] = b[:, :H]
    b1_ref[...] = b[:, H:]


def _prep_nodes(x, WA, bA, WB, bB, WV, bV, WU, bU):
    half = jax.ShapeDtypeStruct((N, H), _f32)
    full = jax.ShapeDtypeStruct((N, D), _f32)
    return pl.pallas_call(
        _prep_nodes_body,
        out_shape=[full, full, half, half, full],
    )(x, WA, bA, WB, bB, WV, bV, WU, bU)


_CEB = 2000


def _prep_ce_body(ea2_ref, we_ref, be_ref, wc_ref, bc_ref, c0_ref, c1_ref):
    wec = jnp.dot(we_ref[...], wc_ref[...], preferred_element_type=_f32)
    bec = jnp.dot(be_ref[...], wc_ref[...], preferred_element_type=_f32) + bc_ref[...]
    ea_ev = ea2_ref[:, :DE]
    ea_od = ea2_ref[:, DE:]
    for h, out_ref in ((0, c0_ref), (1, c1_ref)):
        w_h = wec[:, h * H:(h + 1) * H]
        b_h = bec[:, h * H:(h + 1) * H]
        out_ref[...] = jnp.concatenate(
            [jnp.dot(ea_ev, w_h, preferred_element_type=_f32) + b_h,
             jnp.dot(ea_od, w_h, preferred_element_type=_f32) + b_h],
            axis=1)


def _prep_ce(edge_attr, We, be, WC, bC):
    ea2 = edge_attr.reshape(E // 2, 2 * DE)
    pan = jax.ShapeDtypeStruct((E // 2, D), _f32)
    return pl.pallas_call(
        _prep_ce_body,
        grid=(E // 2 // _CEB,),
        in_specs=[
            pl.BlockSpec((_CEB, 2 * DE), lambda i: (i, 0)),
            pl.BlockSpec((DE, D), lambda i: (0, 0)),
            pl.BlockSpec((1, D), lambda i: (0, 0)),
            pl.BlockSpec((D, D), lambda i: (0, 0)),
            pl.BlockSpec((1, D), lambda i: (0, 0)),
        ],
        out_specs=[
            pl.BlockSpec((_CEB, D), lambda i: (i, 0)),
            pl.BlockSpec((_CEB, D), lambda i: (i, 0)),
        ],
        out_shape=[pan, pan],
    )(ea2, We, be, WC, bC)


def _sc_edge_body(src_h, dst_h, sv0_h, sv1_h, b0_h, b1_h, ce0_h, ce1_h,
                  out0_h, out1_h,
                  i0, i1, i2, i3,
                  gsv0, gsv1, gb0, gb1, gc0, gc1,
                  o0, o1, acc,
                  is0, is1, is2, is3, gs0, gs1, ss0, ss1):
    sid = lax.axis_index("s")
    cid = lax.axis_index("c")
    I = [i0, i1, i2, i3]
    GSV, GB, GC = [gsv0, gsv1], [gb0, gb1], [gc0, gc1]
    O = [o0, o1]
    IS = [is0, is1, is2, is3]
    GS = [gs0, gs1]
    SS = [ss0, ss1]
    irow0 = sid // 2
    icol = H * (sid % 2)
    nch = BASECH + jnp.where(sid < EXTRA, 1, 0)

    def _base(jj):
        return (sid + jj * NSUB) * K

    zv = jnp.zeros((16,), _f32)

    def _zrow(i, carry):
        for g in range(8):
            o0[i, pl.ds(g * 16, 16)] = zv
        return carry

    lax.fori_loop(0, K, _zrow, 0)
    for rep in range(STRIPE // K):
        pltpu.sync_copy(o0, acc.at[pl.ds(sid * STRIPE + rep * K, K)])
    _ztail = STRIPE - (STRIPE // K) * K
    if _ztail:
        pltpu.sync_copy(o0.at[pl.ds(0, _ztail)],
                        acc.at[pl.ds(sid * STRIPE + (STRIPE // K) * K,
                                     _ztail)])

    @pl.when(sid == NSUB - 1)
    def _():
        pltpu.sync_copy(o0.at[pl.ds(0, N - NSUB * STRIPE)],
                        acc.at[pl.ds(NSUB * STRIPE, N - NSUB * STRIPE)])

    plsc.subcore_barrier()

    def _run(sv_h, bt_h, cep_h, out_h):
        def _idx_copies(ri, jj):
            row = irow0 + jj * 8
            return (
                pltpu.make_async_copy(
                    src_h.at[pl.ds(row, 1), pl.ds(icol, K)],
                    I[ri].at[pl.ds(0, 1)], IS[ri]),
                pltpu.make_async_copy(
                    dst_h.at[pl.ds(row, 1), pl.ds(icol, K)],
                    I[ri].at[pl.ds(1, 1)], IS[ri]),
            )

        def _gather_copies(ri, gr, jj):
            return (
                pltpu.make_async_copy(sv_h.at[I[ri].at[0]], GSV[gr], GS[gr]),
                pltpu.make_async_copy(bt_h.at[I[ri].at[1]], GB[gr], GS[gr]),
                pltpu.make_async_copy(
                    cep_h.at[pl.ds(_base(jj) // 2, K // 2)],
                    GC[gr], GS[gr]),
            )

        def _scatter_copy(gr, ri):
            return pltpu.make_async_copy(O[gr], acc.at[I[ri].at[1]], SS[gr])

        def _compute(gr):
            gsv, gb, gc, ob = GSV[gr], GB[gr], GC[gr], O[gr]

            @plsc.parallel_loop(0, K // 2, unroll=2)
            def _pair(i2):
                for half in range(2):
                    i = 2 * i2 + half
                    for g in range(4):
                        sl = pl.ds(g * 16, 16)
                        e = (gsv[i, sl] + gb[i, sl]
                             + gc[i2, pl.ds(half * H + g * 16, 16)])
                        sg = 1.0 / (1.0 + jnp.exp(-e))
                        ob[i, pl.ds(g * 16, 16)] = (
                            sg * gsv[i, pl.ds(H + g * 16, 16)])
                        ob[i, pl.ds(H + g * 16, 16)] = sg

        def _stage(jj, r4):
            r2 = r4 % 2
            for cp in _gather_copies(r4, r2, jj):
                cp.wait()

            @pl.when(jj + 1 < nch)
            def _():
                for cp in _idx_copies((r4 + 1) % 4, jj + 1):
                    cp.wait()
                for cp in _gather_copies((r4 + 1) % 4, 1 - r2, jj + 1):
                    cp.start()

            @pl.when(jj >= 2)
            def _():
                _scatter_copy(r2, (r4 + 2) % 4).wait()

            @pl.when(jj + 2 < nch)
            def _():
                for cp in _idx_copies((r4 + 2) % 4, jj + 2):
                    cp.start()

            _compute(r2)
            _scatter_copy(r2, r4).start(add=True)

        for cp in _idx_copies(0, 0):
            cp.start()
        for cp in _idx_copies(0, 0):
            cp.wait()
        for cp in _gather_copies(0, 0, 0):
            cp.start()
        for cp in _idx_copies(1, 1):
            cp.start()

        def _group(g, carry):
            for r4 in range(4):
                _stage(g * 4 + r4, r4)
            return carry

        lax.fori_loop(0, GROUPS, _group, 0)

        @pl.when(nch > BASECH)
        def _():
            _stage(BASECH, 0)

        _scatter_copy(0, 0).wait()
        _scatter_copy(1, 1).wait()

        plsc.subcore_barrier()
        pltpu.sync_copy(acc.at[pl.ds(sid * STRIPE, STRIPE)],
                        out_h.at[pl.ds(sid * STRIPE, STRIPE)])

        @pl.when(sid == NSUB - 1)
        def _():
            pltpu.sync_copy(acc.at[pl.ds(NSUB * STRIPE, N - NSUB * STRIPE)],
                            out_h.at[pl.ds(NSUB * STRIPE, N - NSUB * STRIPE)])

    @pl.when(cid == 0)
    def _():
        _run(sv0_h, b0_h, ce0_h, out0_h)

    @pl.when(cid == 1)
    def _():
        _run(sv1_h, b1_h, ce1_h, out1_h)


def _sc_edge(src2d, dst2d, sv0, sv1, b0, b1, ce0, ce1):
    mesh = plsc.VectorSubcoreMesh(core_axis_name="c", subcore_axis_name="s",
                                  num_cores=2, num_subcores=NSUB)
    out = jax.ShapeDtypeStruct((N, D), _f32)
    fn = pl.kernel(
        _sc_edge_body,
        out_type=[out, out],
        mesh=mesh,
        compiler_params=pltpu.CompilerParams(use_tc_tiling_on_sc=False),
        scratch_types=(
            [pltpu.VMEM((2, K), jnp.int32)] * 4
            + [pltpu.VMEM((K, D), _f32)] * 2
            + [pltpu.VMEM((K, H), _f32)] * 2
            + [pltpu.VMEM((K // 2, D), _f32)] * 2
            + [pltpu.VMEM((K, D), _f32)] * 2
            + [pltpu.VMEM_SHARED((N, D), _f32)]
            + [pltpu.SemaphoreType.DMA] * 8
        ),
    )
    return fn(src2d, dst2d, sv0, sv1, b0, b1, ce0, ce1)


def _bn_in(h, g, b):
    m = jnp.mean(h, axis=0, keepdims=True)
    v = jnp.mean((h - m) ** 2, axis=0, keepdims=True)
    return g * (h - m) / jnp.sqrt(v + 1e-5) + b


def _post_body(x_ref, u_ref, o0_ref, o1_ref, gn_ref, bn_ref,
               wf1_ref, bf1_ref, wf2_ref, bf2_ref,
               g1_ref, be1_ref, g2_ref, be2_ref, out_ref):
    o0 = o0_ref[...]
    o1 = o1_ref[...]
    num = jnp.concatenate([o0[:, :H], o1[:, :H]], axis=1)
    den = jnp.concatenate([o0[:, H:], o1[:, H:]], axis=1)
    h = u_ref[...] + num / (den + 1e-6)
    h = _bn_in(h, gn_ref[...], bn_ref[...])
    h = jnp.maximum(h, 0.0)
    h = x_ref[...] + h
    pre = h
    h2 = _bn_in(h, g1_ref[...], be1_ref[...])
    h2 = jnp.dot(h2, wf1_ref[...], preferred_element_type=_f32) + bf1_ref[...]
    h2 = jnp.maximum(h2, 0.0)
    h2 = jnp.dot(h2, wf2_ref[...], preferred_element_type=_f32) + bf2_ref[...]
    h = pre + h2
    out_ref[...] = _bn_in(h, g2_ref[...], be2_ref[...])


def _post(x, u, o0, o1, g_node, b_node, Wff1, bff1, Wff2, bff2,
          g1, be1, g2, be2):
    return pl.pallas_call(
        _post_body,
        out_shape=jax.ShapeDtypeStruct((N, D), _f32),
    )(x, u, o0, o1, g_node, b_node, Wff1, bff1, Wff2, bff2, g1, be1, g2, be2)


def kernel(x, edge_index, edge_attr, WA, bA, WB, bB, WC, bC, WU, bU, WV, bV,
           We, be, g_node, b_node, Wff1, bff1, Wff2, bff2, g1, be1, g2, be2):
    eidx = edge_index.astype(jnp.int32)
    src2d = eidx[0].reshape(E // D, D)
    dst2d = eidx[1].reshape(E // D, D)
    r = lambda t: t.reshape(1, -1)
    sv0, sv1, b0, b1, u = _prep_nodes(
        x, WA, r(bA), WB, r(bB), WV, r(bV), WU, r(bU))
    ce0, ce1 = _prep_ce(edge_attr, We, r(be), WC, r(bC))
    o0, o1 = _sc_edge(src2d, dst2d, sv0, sv1, b0, b1, ce0, ce1)
    return _post(x, u, o0, o1, r(g_node), r(b_node), Wff1, r(bff1),
                 Wff2, r(bff2), r(g1), r(be1), r(g2), r(be2))


# speedup vs baseline: 1.2186x; 1.2186x over previous
import functools

import jax
import jax.numpy as jnp
from jax import lax
from jax.experimental import pallas as pl
from jax.experimental.pallas import tpu as pltpu
from jax.experimental.pallas import tpu_sc as plsc

N = 10000
E = 320000
D = 128
H = D // 2
NSUB = 16
K = 64
NCHT = E // K
BASECH = NCHT // NSUB
EXTRA = NCHT % NSUB
GROUPS = BASECH // 4
STRIPE = 624

_f32 = jnp.float32


def _pack_bf16_pair(lo, hi):
    ulo = lax.bitcast_convert_type(lo, jnp.uint32)
    uhi = lax.bitcast_convert_type(hi, jnp.uint32)
    rlo = (ulo + jnp.uint32(0x8000)) >> jnp.uint32(16)
    rhi = (uhi + jnp.uint32(0x8000)) & jnp.uint32(0xFFFF0000)
    return lax.bitcast_convert_type(rlo | rhi, jnp.int32)


def _prep_nodes_body(x_ref, wa_ref, ba_ref, wb_ref, bb_ref, wv_ref, bv_ref,
                     wu_ref, bu_ref,
                     sv0_ref, sv1_ref, b0_ref, b1_ref, u_ref):
    xv = x_ref[...]
    a = jnp.dot(xv, wa_ref[...], preferred_element_type=_f32) + ba_ref[...]
    b = jnp.dot(xv, wb_ref[...], preferred_element_type=_f32) + bb_ref[...]
    v = jnp.dot(xv, wv_ref[...], preferred_element_type=_f32) + bv_ref[...]
    u_ref[...] = jnp.dot(xv, wu_ref[...], preferred_element_type=_f32) + bu_ref[...]
    sv0_ref[...] = _pack_bf16_pair(a[:, :H], v[:, :H])
    sv1_ref[...] = _pack_bf16_pair(a[:, H:], v[:, H:])
    b0_ref[...] = b[:, :H]
    b1_ref[...] = b[:, H:]


def _prep_nodes(x, WA, bA, WB, bB, WV, bV, WU, bU):
    half = jax.ShapeDtypeStruct((N, H), _f32)
    halfi = jax.ShapeDtypeStruct((N, H), jnp.int32)
    full = jax.ShapeDtypeStruct((N, D), _f32)
    return pl.pallas_call(
        _prep_nodes_body,
        out_shape=[halfi, halfi, half, half, full],
    )(x, WA, bA, WB, bB, WV, bV, WU, bU)


_CEB = 4000


def _prep_ce_body(ea_ref, we_ref, be_ref, wc_ref, bc_ref, ce_ref):
    wec = jnp.dot(we_ref[...], wc_ref[...], preferred_element_type=_f32)
    bec = jnp.dot(be_ref[...], wc_ref[...], preferred_element_type=_f32) + bc_ref[...]
    ce_ref[...] = jnp.dot(ea_ref[...], wec, preferred_element_type=_f32) + bec


def _prep_ce(edge_attr, We, be, WC, bC):
    de = edge_attr.shape[1]
    return pl.pallas_call(
        _prep_ce_body,
        grid=(E // _CEB,),
        in_specs=[
            pl.BlockSpec((_CEB, de), lambda i: (i, 0)),
            pl.BlockSpec((de, D), lambda i: (0, 0)),
            pl.BlockSpec((1, D), lambda i: (0, 0)),
            pl.BlockSpec((D, D), lambda i: (0, 0)),
            pl.BlockSpec((1, D), lambda i: (0, 0)),
        ],
        out_specs=pl.BlockSpec((_CEB, D), lambda i: (i, 0)),
        out_shape=jax.ShapeDtypeStruct((E, D), _f32),
    )(edge_attr, We, be, WC, bC)


def _sc_edge_body(src_h, dst_h, sv0_h, sv1_h, b0_h, b1_h, ce_h,
                  out0_h, out1_h,
                  i0, i1, i2, i3,
                  gsv0, gsv1, gb0, gb1, gc0, gc1,
                  o0, o1, acc,
                  is0, is1, is2, is3, gs0, gs1, ss0, ss1):
    sid = lax.axis_index("s")
    cid = lax.axis_index("c")
    I = [i0, i1, i2, i3]
    GSV, GB, GC = [gsv0, gsv1], [gb0, gb1], [gc0, gc1]
    O = [o0, o1]
    IS = [is0, is1, is2, is3]
    GS = [gs0, gs1]
    SS = [ss0, ss1]
    irow0 = sid // 2
    icol = H * (sid % 2)
    nch = BASECH + jnp.where(sid < EXTRA, 1, 0)

    def _base(jj):
        return (sid + jj * NSUB) * K

    zv = jnp.zeros((16,), _f32)

    def _zrow(i, carry):
        for g in range(8):
            o0[i, pl.ds(g * 16, 16)] = zv
        return carry

    lax.fori_loop(0, K, _zrow, 0)
    for rep in range(STRIPE // K):
        pltpu.sync_copy(o0, acc.at[pl.ds(sid * STRIPE + rep * K, K)])
    _ztail = STRIPE - (STRIPE // K) * K
    if _ztail:
        pltpu.sync_copy(o0.at[pl.ds(0, _ztail)],
                        acc.at[pl.ds(sid * STRIPE + (STRIPE // K) * K,
                                     _ztail)])

    @pl.when(sid == NSUB - 1)
    def _():
        pltpu.sync_copy(o0.at[pl.ds(0, N - NSUB * STRIPE)],
                        acc.at[pl.ds(NSUB * STRIPE, N - NSUB * STRIPE)])

    plsc.subcore_barrier()

    def _run(sv_h, bt_h, hcol, out_h):
        def _idx_copies(ri, jj):
            row = irow0 + jj * 8
            return (
                pltpu.make_async_copy(
                    src_h.at[pl.ds(row, 1), pl.ds(icol, K)],
                    I[ri].at[pl.ds(0, 1)], IS[ri]),
                pltpu.make_async_copy(
                    dst_h.at[pl.ds(row, 1), pl.ds(icol, K)],
                    I[ri].at[pl.ds(1, 1)], IS[ri]),
            )

        def _gather_copies(ri, gr, jj):
            return (
                pltpu.make_async_copy(sv_h.at[I[ri].at[0]], GSV[gr], GS[gr]),
                pltpu.make_async_copy(bt_h.at[I[ri].at[1]], GB[gr], GS[gr]),
                pltpu.make_async_copy(
                    ce_h.at[pl.ds(_base(jj), K), pl.ds(hcol, H)],
                    GC[gr], GS[gr]),
            )

        def _scatter_copy(gr, ri):
            return pltpu.make_async_copy(O[gr], acc.at[I[ri].at[1]], SS[gr])

        def _compute(gr):
            gsv, gb, gc, ob = GSV[gr], GB[gr], GC[gr], O[gr]

            msk = jnp.full((16,), -65536, jnp.int32)
            sh16 = jnp.full((16,), 16, jnp.int32)

            @plsc.parallel_loop(0, K, unroll=2)
            def _edge(i):
                for g in range(4):
                    sl = pl.ds(g * 16, 16)
                    w = gsv[i, sl]
                    a = plsc.bitcast(lax.shift_left(w, sh16), _f32)
                    vv = plsc.bitcast(lax.bitwise_and(w, msk), _f32)
                    e = a + gb[i, sl] + gc[i, sl]
                    sg = 1.0 / (1.0 + jnp.exp(-e))
                    ob[i, pl.ds(g * 16, 16)] = sg * vv
                    ob[i, pl.ds(H + g * 16, 16)] = sg

        def _stage(jj, r4):
            r2 = r4 % 2
            for cp in _gather_copies(r4, r2, jj):
                cp.wait()

            @pl.when(jj + 1 < nch)
            def _():
                for cp in _idx_copies((r4 + 1) % 4, jj + 1):
                    cp.wait()
                for cp in _gather_copies((r4 + 1) % 4, 1 - r2, jj + 1):
                    cp.start()

            @pl.when(jj >= 2)
            def _():
                _scatter_copy(r2, (r4 + 2) % 4).wait()

            @pl.when(jj + 2 < nch)
            def _():
                for cp in _idx_copies((r4 + 2) % 4, jj + 2):
                    cp.start()

            _compute(r2)
            _scatter_copy(r2, r4).start(add=True)

        for cp in _idx_copies(0, 0):
            cp.start()
        for cp in _idx_copies(0, 0):
            cp.wait()
        for cp in _gather_copies(0, 0, 0):
            cp.start()
        for cp in _idx_copies(1, 1):
            cp.start()

        def _group(g, carry):
            for r4 in range(4):
                _stage(g * 4 + r4, r4)
            return carry

        lax.fori_loop(0, GROUPS, _group, 0)

        @pl.when(nch > BASECH)
        def _():
            _stage(BASECH, 0)

        _scatter_copy(0, 0).wait()
        _scatter_copy(1, 1).wait()

        plsc.subcore_barrier()
        pltpu.sync_copy(acc.at[pl.ds(sid * STRIPE, STRIPE)],
                        out_h.at[pl.ds(sid * STRIPE, STRIPE)])

        @pl.when(sid == NSUB - 1)
        def _():
            pltpu.sync_copy(acc.at[pl.ds(NSUB * STRIPE, N - NSUB * STRIPE)],
                            out_h.at[pl.ds(NSUB * STRIPE, N - NSUB * STRIPE)])

    @pl.when(cid == 0)
    def _():
        _run(sv0_h, b0_h, 0, out0_h)

    @pl.when(cid == 1)
    def _():
        _run(sv1_h, b1_h, H, out1_h)


def _sc_edge(src2d, dst2d, sv0, sv1, b0, b1, ce):
    mesh = plsc.VectorSubcoreMesh(core_axis_name="c", subcore_axis_name="s",
                                  num_cores=2, num_subcores=NSUB)
    out = jax.ShapeDtypeStruct((N, D), _f32)
    fn = pl.kernel(
        _sc_edge_body,
        out_type=[out, out],
        mesh=mesh,
        compiler_params=pltpu.CompilerParams(use_tc_tiling_on_sc=False,
                                             needs_layout_passes=False),
        scratch_types=(
            [pltpu.VMEM((2, K), jnp.int32)] * 4
            + [pltpu.VMEM((K, H), jnp.int32)] * 2
            + [pltpu.VMEM((K, H), _f32)] * 2
            + [pltpu.VMEM((K, H), _f32)] * 2
            + [pltpu.VMEM((K, D), _f32)] * 2
            + [pltpu.VMEM_SHARED((N, D), _f32)]
            + [pltpu.SemaphoreType.DMA] * 8
        ),
    )
    return fn(src2d, dst2d, sv0, sv1, b0, b1, ce)


def _bn_in(h, g, b):
    m = jnp.mean(h, axis=0, keepdims=True)
    v = jnp.mean((h - m) ** 2, axis=0, keepdims=True)
    return g * (h - m) / jnp.sqrt(v + 1e-5) + b


def _post_body(x_ref, u_ref, o0_ref, o1_ref, gn_ref, bn_ref,
               wf1_ref, bf1_ref, wf2_ref, bf2_ref,
               g1_ref, be1_ref, g2_ref, be2_ref, out_ref):
    o0 = o0_ref[...]
    o1 = o1_ref[...]
    num = jnp.concatenate([o0[:, :H], o1[:, :H]], axis=1)
    den = jnp.concatenate([o0[:, H:], o1[:, H:]], axis=1)
    h = u_ref[...] + num / (den + 1e-6)
    h = _bn_in(h, gn_ref[...], bn_ref[...])
    h = jnp.maximum(h, 0.0)
    h = x_ref[...] + h
    pre = h
    h2 = _bn_in(h, g1_ref[...], be1_ref[...])
    h2 = jnp.dot(h2, wf1_ref[...], preferred_element_type=_f32) + bf1_ref[...]
    h2 = jnp.maximum(h2, 0.0)
    h2 = jnp.dot(h2, wf2_ref[...], preferred_element_type=_f32) + bf2_ref[...]
    h = pre + h2
    out_ref[...] = _bn_in(h, g2_ref[...], be2_ref[...])


def _post(x, u, o0, o1, g_node, b_node, Wff1, bff1, Wff2, bff2,
          g1, be1, g2, be2):
    return pl.pallas_call(
        _post_body,
        out_shape=jax.ShapeDtypeStruct((N, D), _f32),
    )(x, u, o0, o1, g_node, b_node, Wff1, bff1, Wff2, bff2, g1, be1, g2, be2)


def kernel(x, edge_index, edge_attr, WA, bA, WB, bB, WC, bC, WU, bU, WV, bV,
           We, be, g_node, b_node, Wff1, bff1, Wff2, bff2, g1, be1, g2, be2):
    eidx = edge_index.astype(jnp.int32)
    src2d = eidx[0].reshape(E // D, D)
    dst2d = eidx[1].reshape(E // D, D)
    r = lambda t: t.reshape(1, -1)
    sv0, sv1, b0, b1, u = _prep_nodes(
        x, WA, r(bA), WB, r(bB), WV, r(bV), WU, r(bU))
    ce = _prep_ce(edge_attr, We, r(be), WC, r(bC))
    o0, o1 = _sc_edge(src2d, dst2d, sv0, sv1, b0, b1, ce)
    return _post(x, u, o0, o1, r(g_node), r(b_node), Wff1, r(bff1),
                 Wff2, r(bff2), r(g1), r(be1), r(g2), r(be2))


# speedup vs baseline: 1.2904x; 1.0589x over previous
import functools

import jax
import jax.numpy as jnp
from jax import lax
from jax.experimental import pallas as pl
from jax.experimental.pallas import tpu as pltpu
from jax.experimental.pallas import tpu_sc as plsc

N = 10000
E = 320000
D = 128
H = D // 2
NSUB = 16
K = 64
NCHT = E // K
BASECH = NCHT // NSUB
EXTRA = NCHT % NSUB
GROUPS = BASECH // 4
STRIPE = 624

_f32 = jnp.float32


def _pack_bf16_pair(lo, hi):
    ulo = lax.bitcast_convert_type(lo, jnp.uint32)
    uhi = lax.bitcast_convert_type(hi, jnp.uint32)
    rlo = (ulo + jnp.uint32(0x8000)) >> jnp.uint32(16)
    rhi = (uhi + jnp.uint32(0x8000)) & jnp.uint32(0xFFFF0000)
    return lax.bitcast_convert_type(rlo | rhi, jnp.int32)


def _prep_nodes_body(x_ref, wa_ref, ba_ref, wb_ref, bb_ref, wv_ref, bv_ref,
                     wu_ref, bu_ref,
                     sv0_ref, sv1_ref, b0_ref, b1_ref, u_ref):
    xv = x_ref[...]
    a = jnp.dot(xv, wa_ref[...], preferred_element_type=_f32) + ba_ref[...]
    b = jnp.dot(xv, wb_ref[...], preferred_element_type=_f32) + bb_ref[...]
    v = jnp.dot(xv, wv_ref[...], preferred_element_type=_f32) + bv_ref[...]
    u_ref[...] = jnp.dot(xv, wu_ref[...], preferred_element_type=_f32) + bu_ref[...]
    sv0_ref[...] = _pack_bf16_pair(a[:, :H], v[:, :H])
    sv1_ref[...] = _pack_bf16_pair(a[:, H:], v[:, H:])
    q = H // 2
    b0_ref[...] = _pack_bf16_pair(b[:, :q], b[:, q:H])
    b1_ref[...] = _pack_bf16_pair(b[:, H:H + q], b[:, H + q:])


def _prep_nodes(x, WA, bA, WB, bB, WV, bV, WU, bU):
    halfi = jax.ShapeDtypeStruct((N, H), jnp.int32)
    quarti = jax.ShapeDtypeStruct((N, H // 2), jnp.int32)
    full = jax.ShapeDtypeStruct((N, D), _f32)
    return pl.pallas_call(
        _prep_nodes_body,
        out_shape=[halfi, halfi, quarti, quarti, full],
    )(x, WA, bA, WB, bB, WV, bV, WU, bU)


_CEB = 4000


def _prep_ce_body(ea_ref, we_ref, be_ref, wc_ref, bc_ref, ce_ref):
    wec = jnp.dot(we_ref[...], wc_ref[...], preferred_element_type=_f32)
    bec = jnp.dot(be_ref[...], wc_ref[...], preferred_element_type=_f32) + bc_ref[...]
    ce_ref[...] = jnp.dot(ea_ref[...], wec, preferred_element_type=_f32) + bec


def _prep_ce(edge_attr, We, be, WC, bC):
    de = edge_attr.shape[1]
    return pl.pallas_call(
        _prep_ce_body,
        grid=(E // _CEB,),
        in_specs=[
            pl.BlockSpec((_CEB, de), lambda i: (i, 0)),
            pl.BlockSpec((de, D), lambda i: (0, 0)),
            pl.BlockSpec((1, D), lambda i: (0, 0)),
            pl.BlockSpec((D, D), lambda i: (0, 0)),
            pl.BlockSpec((1, D), lambda i: (0, 0)),
        ],
        out_specs=pl.BlockSpec((_CEB, D), lambda i: (i, 0)),
        out_shape=jax.ShapeDtypeStruct((E, D), _f32),
    )(edge_attr, We, be, WC, bC)


def _sc_edge_body(src_h, dst_h, sv0_h, sv1_h, b0_h, b1_h, ce_h,
                  out0_h, out1_h,
                  i0, i1, i2, i3,
                  gsv0, gsv1, gb0, gb1, gc0, gc1,
                  o0, o1, acc,
                  is0, is1, is2, is3, gs0, gs1, ss0, ss1):
    sid = lax.axis_index("s")
    cid = lax.axis_index("c")
    I = [i0, i1, i2, i3]
    GSV, GB, GC = [gsv0, gsv1], [gb0, gb1], [gc0, gc1]
    O = [o0, o1]
    IS = [is0, is1, is2, is3]
    GS = [gs0, gs1]
    SS = [ss0, ss1]
    irow0 = sid // 2
    icol = H * (sid % 2)
    nch = BASECH + jnp.where(sid < EXTRA, 1, 0)

    def _base(jj):
        return (sid + jj * NSUB) * K

    zv = jnp.zeros((16,), _f32)

    def _zrow(i, carry):
        for g in range(8):
            o0[i, pl.ds(g * 16, 16)] = zv
        return carry

    lax.fori_loop(0, K, _zrow, 0)
    for rep in range(STRIPE // K):
        pltpu.sync_copy(o0, acc.at[pl.ds(sid * STRIPE + rep * K, K)])
    _ztail = STRIPE - (STRIPE // K) * K
    if _ztail:
        pltpu.sync_copy(o0.at[pl.ds(0, _ztail)],
                        acc.at[pl.ds(sid * STRIPE + (STRIPE // K) * K,
                                     _ztail)])

    @pl.when(sid == NSUB - 1)
    def _():
        pltpu.sync_copy(o0.at[pl.ds(0, N - NSUB * STRIPE)],
                        acc.at[pl.ds(NSUB * STRIPE, N - NSUB * STRIPE)])

    plsc.subcore_barrier()

    def _run(sv_h, bt_h, hcol, out_h):
        def _idx_copies(ri, jj):
            row = irow0 + jj * 8
            return (
                pltpu.make_async_copy(
                    src_h.at[pl.ds(row, 1), pl.ds(icol, K)],
                    I[ri].at[pl.ds(0, 1)], IS[ri]),
                pltpu.make_async_copy(
                    dst_h.at[pl.ds(row, 1), pl.ds(icol, K)],
                    I[ri].at[pl.ds(1, 1)], IS[ri]),
            )

        def _gather_copies(ri, gr, jj):
            return (
                pltpu.make_async_copy(sv_h.at[I[ri].at[0]], GSV[gr], GS[gr]),
                pltpu.make_async_copy(bt_h.at[I[ri].at[1]], GB[gr], GS[gr]),
                pltpu.make_async_copy(
                    ce_h.at[pl.ds(_base(jj), K), pl.ds(hcol, H)],
                    GC[gr], GS[gr]),
            )

        def _scatter_copy(gr, ri):
            return pltpu.make_async_copy(O[gr], acc.at[I[ri].at[1]], SS[gr])

        def _compute(gr):
            gsv, gb, gc, ob = GSV[gr], GB[gr], GC[gr], O[gr]

            msk = jnp.full((16,), -65536, jnp.int32)
            sh16 = jnp.full((16,), 16, jnp.int32)

            def _lo(w):
                return plsc.bitcast(lax.shift_left(w, sh16), _f32)

            def _hi(w):
                return plsc.bitcast(lax.bitwise_and(w, msk), _f32)

            @plsc.parallel_loop(0, K, unroll=2)
            def _edge(i):
                for g2 in range(2):
                    wb = gb[i, pl.ds(g2 * 16, 16)]
                    for g, bval in ((g2, _lo(wb)), (g2 + 2, _hi(wb))):
                        sl = pl.ds(g * 16, 16)
                        w = gsv[i, sl]
                        e = _lo(w) + bval + gc[i, sl]
                        sg = 1.0 / (1.0 + jnp.exp(-e))
                        ob[i, pl.ds(g * 16, 16)] = sg * _hi(w)
                        ob[i, pl.ds(H + g * 16, 16)] = sg

        def _stage(jj, r4):
            r2 = r4 % 2
            for cp in _gather_copies(r4, r2, jj):
                cp.wait()

            @pl.when(jj + 1 < nch)
            def _():
                for cp in _idx_copies((r4 + 1) % 4, jj + 1):
                    cp.wait()
                for cp in _gather_copies((r4 + 1) % 4, 1 - r2, jj + 1):
                    cp.start()

            @pl.when(jj >= 2)
            def _():
                _scatter_copy(r2, (r4 + 2) % 4).wait()

            @pl.when(jj + 2 < nch)
            def _():
                for cp in _idx_copies((r4 + 2) % 4, jj + 2):
                    cp.start()

            _compute(r2)
            _scatter_copy(r2, r4).start(add=True)

        for cp in _idx_copies(0, 0):
            cp.start()
        for cp in _idx_copies(0, 0):
            cp.wait()
        for cp in _gather_copies(0, 0, 0):
            cp.start()
        for cp in _idx_copies(1, 1):
            cp.start()

        def _group(g, carry):
            for r4 in range(4):
                _stage(g * 4 + r4, r4)
            return carry

        lax.fori_loop(0, GROUPS, _group, 0)

        @pl.when(nch > BASECH)
        def _():
            _stage(BASECH, 0)

        _scatter_copy(0, 0).wait()
        _scatter_copy(1, 1).wait()

        plsc.subcore_barrier()
        pltpu.sync_copy(acc.at[pl.ds(sid * STRIPE, STRIPE)],
                        out_h.at[pl.ds(sid * STRIPE, STRIPE)])

        @pl.when(sid == NSUB - 1)
        def _():
            pltpu.sync_copy(acc.at[pl.ds(NSUB * STRIPE, N - NSUB * STRIPE)],
                            out_h.at[pl.ds(NSUB * STRIPE, N - NSUB * STRIPE)])

    @pl.when(cid == 0)
    def _():
        _run(sv0_h, b0_h, 0, out0_h)

    @pl.when(cid == 1)
    def _():
        _run(sv1_h, b1_h, H, out1_h)


def _sc_edge(src2d, dst2d, sv0, sv1, b0, b1, ce):
    mesh = plsc.VectorSubcoreMesh(core_axis_name="c", subcore_axis_name="s",
                                  num_cores=2, num_subcores=NSUB)
    out = jax.ShapeDtypeStruct((N, D), _f32)
    fn = pl.kernel(
        _sc_edge_body,
        out_type=[out, out],
        mesh=mesh,
        compiler_params=pltpu.CompilerParams(use_tc_tiling_on_sc=False,
                                             needs_layout_passes=False),
        scratch_types=(
            [pltpu.VMEM((2, K), jnp.int32)] * 4
            + [pltpu.VMEM((K, H), jnp.int32)] * 2
            + [pltpu.VMEM((K, H // 2), jnp.int32)] * 2
            + [pltpu.VMEM((K, H), _f32)] * 2
            + [pltpu.VMEM((K, D), _f32)] * 2
            + [pltpu.VMEM_SHARED((N, D), _f32)]
            + [pltpu.SemaphoreType.DMA] * 8
        ),
    )
    return fn(src2d, dst2d, sv0, sv1, b0, b1, ce)


def _bn_in(h, g, b):
    m = jnp.mean(h, axis=0, keepdims=True)
    v = jnp.mean((h - m) ** 2, axis=0, keepdims=True)
    return g * (h - m) / jnp.sqrt(v + 1e-5) + b


def _post_body(x_ref, u_ref, o0_ref, o1_ref, gn_ref, bn_ref,
               wf1_ref, bf1_ref, wf2_ref, bf2_ref,
               g1_ref, be1_ref, g2_ref, be2_ref, out_ref):
    o0 = o0_ref[...]
    o1 = o1_ref[...]
    num = jnp.concatenate([o0[:, :H], o1[:, :H]], axis=1)
    den = jnp.concatenate([o0[:, H:], o1[:, H:]], axis=1)
    h = u_ref[...] + num / (den + 1e-6)
    h = _bn_in(h, gn_ref[...], bn_ref[...])
    h = jnp.maximum(h, 0.0)
    h = x_ref[...] + h
    pre = h
    h2 = _bn_in(h, g1_ref[...], be1_ref[...])
    h2 = jnp.dot(h2, wf1_ref[...], preferred_element_type=_f32) + bf1_ref[...]
    h2 = jnp.maximum(h2, 0.0)
    h2 = jnp.dot(h2, wf2_ref[...], preferred_element_type=_f32) + bf2_ref[...]
    h = pre + h2
    out_ref[...] = _bn_in(h, g2_ref[...], be2_ref[...])


def _post(x, u, o0, o1, g_node, b_node, Wff1, bff1, Wff2, bff2,
          g1, be1, g2, be2):
    return pl.pallas_call(
        _post_body,
        out_shape=jax.ShapeDtypeStruct((N, D), _f32),
    )(x, u, o0, o1, g_node, b_node, Wff1, bff1, Wff2, bff2, g1, be1, g2, be2)


def kernel(x, edge_index, edge_attr, WA, bA, WB, bB, WC, bC, WU, bU, WV, bV,
           We, be, g_node, b_node, Wff1, bff1, Wff2, bff2, g1, be1, g2, be2):
    eidx = edge_index.astype(jnp.int32)
    src2d = eidx[0].reshape(E // D, D)
    dst2d = eidx[1].reshape(E // D, D)
    r = lambda t: t.reshape(1, -1)
    sv0, sv1, b0, b1, u = _prep_nodes(
        x, WA, r(bA), WB, r(bB), WV, r(bV), WU, r(bU))
    ce = _prep_ce(edge_attr, We, r(be), WC, r(bC))
    o0, o1 = _sc_edge(src2d, dst2d, sv0, sv1, b0, b1, ce)
    return _post(x, u, o0, o1, r(g_node), r(b_node), Wff1, r(bff1),
                 Wff2, r(bff2), r(g1), r(be1), r(g2), r(be2))
